# baseline (speedup 1.0000x reference)
import numpy as np
import jax
import jax.numpy as jnp
from jax import lax
from jax.experimental import pallas as pl
from jax.experimental.pallas import tpu as pltpu

_SIG1 = float(1.0 / (1.0 + np.exp(-1.0)))


def _graph_kernel(corners_ref, centers_ref, mask_ref, x_ref,
                  w1a0_ref, w1b0_ref, b10_ref, w20_ref, b20_ref,
                  w1a1_ref, w1b1_ref, b11_ref, w21_ref, b21_ref,
                  feat_ref, adj_ref, gm_ref, hb_ref):
    N = adj_ref.shape[0]
    E = gm_ref.shape[0]
    K = E // N

    centers = centers_ref[...]
    neg2c = -2.0 * centers
    cnorm = jnp.sum(centers * centers, axis=0, keepdims=True)

    corners = corners_ref[...]
    cross_all = jnp.dot(corners.astype(jnp.bfloat16),
                        neg2c.astype(jnp.bfloat16),
                        preferred_element_type=jnp.float32)
    corner_norms = jnp.sum(corners * corners, axis=1,
                           keepdims=True)
    d2_all = cross_all + corner_norms

    d2min = d2_all[0:N, :]
    for k in range(1, 8):
        d2min = jnp.minimum(d2min, d2_all[k * N:(k + 1) * N, :])

    dist = jnp.sqrt(jnp.maximum(d2min + cnorm, 0.0) + 1e-8)

    row = lax.broadcasted_iota(jnp.int32, (N, N), 0)
    col = lax.broadcasted_iota(jnp.int32, (N, N), 1)
    invalid = (mask_ref[...] == 0.0) | (row == col)
    d = jnp.where(invalid, 1e30, dist)

    col_f = col.astype(jnp.float32)
    for k in range(K):
        rowmin = jnp.min(d, axis=1, keepdims=True)
        cand = jnp.where(d == rowmin, col_f, 1e9)
        rowidx = jnp.min(cand, axis=1, keepdims=True)
        a_k = col_f == rowidx
        d = jnp.where(a_k, jnp.inf, d)
        gm_ref[k * N:(k + 1) * N, :] = a_k.astype(jnp.bfloat16)
    adj = (d == jnp.inf).astype(jnp.float32)
    adj_ref[...] = adj

    ones_col = jnp.ones((N, 1), jnp.float32)
    indeg = lax.dot_general(adj, ones_col, (((0,), (0,)), ((), ())),
                            preferred_element_type=jnp.float32)

    gm = gm_ref[...]
    x0 = x_ref[...]
    x = x0
    layers = (
        (w1a0_ref, w1b0_ref, b10_ref, w20_ref, b20_ref),
        (w1a1_ref, w1b1_ref, b11_ref, w21_ref, b21_ref),
    )
    for (w1a_ref, w1b_ref, b1_ref, w2_ref, b2_ref) in layers:
        xb = x.astype(jnp.bfloat16)
        P = jnp.dot(xb, w1a_ref[...], preferred_element_type=jnp.float32)
        Q = jnp.dot(xb, w1b_ref[...], preferred_element_type=jnp.float32)
        base = (P - Q + b1_ref[...]).astype(jnp.bfloat16)
        GB = jnp.dot(gm, base, preferred_element_type=jnp.float32)
        for k in range(K):
            hb_ref[k * N:(k + 1) * N, :] = jnp.maximum(
                GB[k * N:(k + 1) * N, :] + Q, 0.0).astype(jnp.bfloat16)
        S = lax.dot_general(gm, hb_ref[...], (((0,), (0,)), ((), ())),
                            preferred_element_type=jnp.float32)
        x = _SIG1 * (
            jnp.dot(S.astype(jnp.bfloat16), w2_ref[...],
                    preferred_element_type=jnp.float32)
            + indeg * b2_ref[...]
        )
    feat_ref[...] = x0 + x


def _graph_fused(corners_cm, centers_t, mask3, x, weights):
    B, N, F = x.shape
    M = corners_cm.shape[1]
    K = 8
    E = K * N
    w_specs = [pl.BlockSpec(w.shape, lambda c, i: (0, 0)) for w in weights]
    return pl.pallas_call(
        _graph_kernel,
        out_shape=(
            jax.ShapeDtypeStruct((B, N, F), jnp.float32),
            jax.ShapeDtypeStruct((B, N, N), jnp.float32),
        ),
        grid_spec=pltpu.PrefetchScalarGridSpec(
            num_scalar_prefetch=0,
            grid=(2, B // 2),
            in_specs=[
                pl.BlockSpec((None, M, 3), lambda c, i: (c * (B // 2) + i, 0, 0)),
                pl.BlockSpec((None, 3, N), lambda c, i: (c * (B // 2) + i, 0, 0)),
                pl.BlockSpec((None, 1, N), lambda c, i: (c * (B // 2) + i, 0, 0)),
                pl.BlockSpec((None, N, F), lambda c, i: (c * (B // 2) + i, 0, 0)),
            ] + w_specs,
            out_specs=(
                pl.BlockSpec((None, N, F), lambda c, i: (c * (B // 2) + i, 0, 0)),
                pl.BlockSpec((None, N, N), lambda c, i: (c * (B // 2) + i, 0, 0)),
            ),
            scratch_shapes=[pltpu.VMEM((E, N), jnp.bfloat16),
                            pltpu.VMEM((E, F), jnp.bfloat16)],
        ),
        compiler_params=pltpu.CompilerParams(
            dimension_semantics=("parallel", "arbitrary")),
    )(corners_cm, centers_t, mask3, x, *weights)


def kernel(object_feats, object_mask, bbox_corner, select_feat_idx,
           gc0_w1, gc0_b1, gc0_w2, gc0_b2,
           gc1_w1, gc1_b1, gc1_w2, gc1_b2):
    B, N, F = object_feats.shape
    K = 8

    coord_min = jnp.min(bbox_corner, axis=2)
    coord_max = jnp.max(bbox_corner, axis=2)
    centers = (coord_min + coord_max) / 2.0
    corners_cm = jnp.transpose(bbox_corner, (0, 2, 1, 3)).reshape(B, 8 * N, 3)
    centers_t = jnp.transpose(centers, (0, 2, 1))
    mask3 = object_mask.reshape(B, 1, N)

    H = gc0_w1.shape[1]
    bf = jnp.bfloat16
    weights = (
        gc0_w1[:F].astype(bf), gc0_w1[F:].astype(bf),
        gc0_b1.reshape(1, H), gc0_w2.astype(bf), gc0_b2.reshape(1, H),
        gc1_w1[:F].astype(bf), gc1_w1[F:].astype(bf),
        gc1_b1.reshape(1, H), gc1_w2.astype(bf), gc1_b2.reshape(1, H),
    )
    bbox_feature, adjacent_mat = _graph_fused(
        corners_cm, centers_t, mask3, object_feats, weights)

    b_idx = jnp.arange(B)
    enhanced_feats = bbox_feature[b_idx, select_feat_idx]
    valid_mask = adjacent_mat[b_idx, select_feat_idx] != 0

    num_bins = 6
    out = {
        "object_feats": object_feats,
        "object_mask": object_mask,
        "bbox_corner": bbox_corner,
        "select_feat_idx": select_feat_idx,
        "bbox_feature": bbox_feature,
        "adjacent_mat": adjacent_mat,
        "enhanced_feats": enhanced_feats,
        "valid_mask": valid_mask,
        "edge_index": jnp.zeros((B, 2, N * K), jnp.float32),
        "edge_feature": jnp.zeros((B, N, K, F), jnp.float32),
        "num_edge_source": jnp.zeros((B,), jnp.int32),
        "num_edge_target": jnp.zeros((B,), jnp.int32),
        "edge_orientations": jnp.zeros((B, N * K, num_bins), jnp.float32),
        "edge_distances": jnp.zeros((B, N * K), jnp.float32),
    }
    return out


# speedup vs baseline: 1.0136x; 1.0136x over previous
import numpy as np
import jax
import jax.numpy as jnp
from jax import lax
from jax.experimental import pallas as pl
from jax.experimental.pallas import tpu as pltpu

_SIG1 = float(1.0 / (1.0 + np.exp(-1.0)))
_SPB = 2


def _graph_kernel(corners_ref, centers_ref, mask_ref, x_ref,
                  w1a0_ref, w1b0_ref, b10_ref, w20_ref, b20_ref,
                  w1a1_ref, w1b1_ref, b11_ref, w21_ref, b21_ref,
                  feat_ref, adj_ref, gm_ref, hb_ref):
    N = adj_ref.shape[1]
    E = gm_ref.shape[1]
    K = E // N

    layers = (
        (w1a0_ref, w1b0_ref, b10_ref, w20_ref, b20_ref),
        (w1a1_ref, w1b1_ref, b11_ref, w21_ref, b21_ref),
    )

    for s in range(_SPB):
        centers = centers_ref[s]
        neg2c = -2.0 * centers
        cnorm = jnp.sum(centers * centers, axis=0, keepdims=True)

        corners = corners_ref[s]
        cross_all = jnp.dot(corners.astype(jnp.bfloat16),
                            neg2c.astype(jnp.bfloat16),
                            preferred_element_type=jnp.float32)
        corner_norms = jnp.sum(corners * corners, axis=1,
                               keepdims=True)
        d2_all = cross_all + corner_norms

        d2min = d2_all[0:N, :]
        for k in range(1, 8):
            d2min = jnp.minimum(d2min, d2_all[k * N:(k + 1) * N, :])

        dist = jnp.sqrt(jnp.maximum(d2min + cnorm, 0.0) + 1e-8)

        row = lax.broadcasted_iota(jnp.int32, (N, N), 0)
        col = lax.broadcasted_iota(jnp.int32, (N, N), 1)
        invalid = (mask_ref[s] == 0.0) | (row == col)
        d = jnp.where(invalid, 1e30, dist)

        col_f = col.astype(jnp.float32)
        for k in range(K):
            rowmin = jnp.min(d, axis=1, keepdims=True)
            cand = jnp.where(d == rowmin, col_f, 1e9)
            rowidx = jnp.min(cand, axis=1, keepdims=True)
            a_k = col_f == rowidx
            d = jnp.where(a_k, jnp.inf, d)
            gm_ref[s, k * N:(k + 1) * N, :] = a_k.astype(jnp.bfloat16)
        adj = (d == jnp.inf).astype(jnp.float32)
        adj_ref[s] = adj

        ones_col = jnp.ones((N, 1), jnp.float32)
        indeg = lax.dot_general(adj, ones_col, (((0,), (0,)), ((), ())),
                                preferred_element_type=jnp.float32)

        gm = gm_ref[s]
        x0 = x_ref[s]
        x = x0
        for (w1a_ref, w1b_ref, b1_ref, w2_ref, b2_ref) in layers:
            xb = x.astype(jnp.bfloat16)
            P = jnp.dot(xb, w1a_ref[...], preferred_element_type=jnp.float32)
            Q = jnp.dot(xb, w1b_ref[...], preferred_element_type=jnp.float32)
            base = (P - Q + b1_ref[...]).astype(jnp.bfloat16)
            GB = jnp.dot(gm, base, preferred_element_type=jnp.float32)
            for k in range(K):
                hb_ref[s, k * N:(k + 1) * N, :] = jnp.maximum(
                    GB[k * N:(k + 1) * N, :] + Q, 0.0).astype(jnp.bfloat16)
            S = lax.dot_general(gm, hb_ref[s], (((0,), (0,)), ((), ())),
                                preferred_element_type=jnp.float32)
            x = _SIG1 * (
                jnp.dot(S.astype(jnp.bfloat16), w2_ref[...],
                        preferred_element_type=jnp.float32)
                + indeg * b2_ref[...]
            )
        feat_ref[s] = x0 + x


def _graph_fused(corners_cm, centers_t, mask3, x, weights):
    B, N, F = x.shape
    M = corners_cm.shape[1]
    K = 8
    E = K * N
    S = _SPB
    w_specs = [pl.BlockSpec(w.shape, lambda b: (0, 0)) for w in weights]
    return pl.pallas_call(
        _graph_kernel,
        out_shape=(
            jax.ShapeDtypeStruct((B, N, F), jnp.float32),
            jax.ShapeDtypeStruct((B, N, N), jnp.float32),
        ),
        grid_spec=pltpu.PrefetchScalarGridSpec(
            num_scalar_prefetch=0,
            grid=(B // S,),
            in_specs=[
                pl.BlockSpec((S, M, 3), lambda b: (b, 0, 0)),
                pl.BlockSpec((S, 3, N), lambda b: (b, 0, 0)),
                pl.BlockSpec((S, 1, N), lambda b: (b, 0, 0)),
                pl.BlockSpec((S, N, F), lambda b: (b, 0, 0)),
            ] + w_specs,
            out_specs=(
                pl.BlockSpec((S, N, F), lambda b: (b, 0, 0)),
                pl.BlockSpec((S, N, N), lambda b: (b, 0, 0)),
            ),
            scratch_shapes=[pltpu.VMEM((S, E, N), jnp.bfloat16),
                            pltpu.VMEM((S, E, F), jnp.bfloat16)],
        ),
        compiler_params=pltpu.CompilerParams(dimension_semantics=("parallel",)),
    )(corners_cm, centers_t, mask3, x, *weights)


def kernel(object_feats, object_mask, bbox_corner, select_feat_idx,
           gc0_w1, gc0_b1, gc0_w2, gc0_b2,
           gc1_w1, gc1_b1, gc1_w2, gc1_b2):
    B, N, F = object_feats.shape
    K = 8

    coord_min = jnp.min(bbox_corner, axis=2)
    coord_max = jnp.max(bbox_corner, axis=2)
    centers = (coord_min + coord_max) / 2.0
    corners_cm = jnp.transpose(bbox_corner, (0, 2, 1, 3)).reshape(B, 8 * N, 3)
    centers_t = jnp.transpose(centers, (0, 2, 1))
    mask3 = object_mask.reshape(B, 1, N)

    H = gc0_w1.shape[1]
    bf = jnp.bfloat16
    weights = (
        gc0_w1[:F].astype(bf), gc0_w1[F:].astype(bf),
        gc0_b1.reshape(1, H), gc0_w2.astype(bf), gc0_b2.reshape(1, H),
        gc1_w1[:F].astype(bf), gc1_w1[F:].astype(bf),
        gc1_b1.reshape(1, H), gc1_w2.astype(bf), gc1_b2.reshape(1, H),
    )
    bbox_feature, adjacent_mat = _graph_fused(
        corners_cm, centers_t, mask3, object_feats, weights)

    b_idx = jnp.arange(B)
    enhanced_feats = bbox_feature[b_idx, select_feat_idx]
    valid_mask = adjacent_mat[b_idx, select_feat_idx] != 0

    num_bins = 6
    out = {
        "object_feats": object_feats,
        "object_mask": object_mask,
        "bbox_corner": bbox_corner,
        "select_feat_idx": select_feat_idx,
        "bbox_feature": bbox_feature,
        "adjacent_mat": adjacent_mat,
        "enhanced_feats": enhanced_feats,
        "valid_mask": valid_mask,
        "edge_index": jnp.zeros((B, 2, N * K), jnp.float32),
        "edge_feature": jnp.zeros((B, N, K, F), jnp.float32),
        "num_edge_source": jnp.zeros((B,), jnp.int32),
        "num_edge_target": jnp.zeros((B,), jnp.int32),
        "edge_orientations": jnp.zeros((B, N * K, num_bins), jnp.float32),
        "edge_distances": jnp.zeros((B, N * K), jnp.float32),
    }
    return out


# speedup vs baseline: 1.0160x; 1.0024x over previous
import numpy as np
import jax
import jax.numpy as jnp
from jax import lax
from jax.experimental import pallas as pl
from jax.experimental.pallas import tpu as pltpu

_SIG1 = float(1.0 / (1.0 + np.exp(-1.0)))
_SPB = 2


def _graph_kernel(corners_ref, centers_ref, mask_ref, x_ref,
                  w1a0_ref, w1b0_ref, b10_ref, w20_ref, b20_ref,
                  w1a1_ref, w1b1_ref, b11_ref, w21_ref, b21_ref,
                  feat_ref, adj_ref, gm0_ref, gm1_ref, hb0_ref, hb1_ref):
    N = adj_ref.shape[1]
    E = gm0_ref.shape[0]
    K = E // N

    layers = (
        (w1a0_ref, w1b0_ref, b10_ref, w20_ref, b20_ref),
        (w1a1_ref, w1b1_ref, b11_ref, w21_ref, b21_ref),
    )

    for s, (gm_ref, hb_ref) in enumerate(((gm0_ref, hb0_ref), (gm1_ref, hb1_ref))):
        centers = centers_ref[s]
        neg2c = -2.0 * centers
        cnorm = jnp.sum(centers * centers, axis=0, keepdims=True)

        corners = corners_ref[s]
        cross_all = jnp.dot(corners, neg2c,
                            preferred_element_type=jnp.float32)
        corner_norms = jnp.sum(corners * corners, axis=1,
                               keepdims=True)
        d2_all = cross_all + corner_norms

        d2min = d2_all[0:N, :]
        for k in range(1, 8):
            d2min = jnp.minimum(d2min, d2_all[k * N:(k + 1) * N, :])

        dist = jnp.sqrt(jnp.maximum(d2min + cnorm, 0.0) + 1e-8)

        row = lax.broadcasted_iota(jnp.int32, (N, N), 0)
        col = lax.broadcasted_iota(jnp.int32, (N, N), 1)
        invalid = (mask_ref[s] == 0.0) | (row == col)
        d = jnp.where(invalid, 1e30, dist)

        col_f = col.astype(jnp.float32)
        for k in range(K):
            rowmin = jnp.min(d, axis=1, keepdims=True)
            cand = jnp.where(d == rowmin, col_f, 1e9)
            rowidx = jnp.min(cand, axis=1, keepdims=True)
            a_k = col_f == rowidx
            d = jnp.where(a_k, jnp.inf, d)
            gm_ref[k * N:(k + 1) * N, :] = a_k.astype(jnp.bfloat16)
        adj = (d == jnp.inf).astype(jnp.float32)
        adj_ref[s] = adj

        ones_col = jnp.ones((N, 1), jnp.float32)
        indeg = lax.dot_general(adj, ones_col, (((0,), (0,)), ((), ())),
                                preferred_element_type=jnp.float32)

        gm = gm_ref[...]
        x0 = x_ref[s]
        x = x0
        for (w1a_ref, w1b_ref, b1_ref, w2_ref, b2_ref) in layers:
            xb = x.astype(jnp.bfloat16)
            P = jnp.dot(xb, w1a_ref[...], preferred_element_type=jnp.float32)
            Q = jnp.dot(xb, w1b_ref[...], preferred_element_type=jnp.float32)
            base = (P - Q + b1_ref[...]).astype(jnp.bfloat16)
            GB = jnp.dot(gm, base, preferred_element_type=jnp.float32)
            for k in range(K):
                hb_ref[k * N:(k + 1) * N, :] = jnp.maximum(
                    GB[k * N:(k + 1) * N, :] + Q, 0.0).astype(jnp.bfloat16)
            S = lax.dot_general(gm, hb_ref[...], (((0,), (0,)), ((), ())),
                                preferred_element_type=jnp.float32)
            x = _SIG1 * (
                jnp.dot(S.astype(jnp.bfloat16), w2_ref[...],
                        preferred_element_type=jnp.float32)
                + indeg * b2_ref[...]
            )
        feat_ref[s] = x0 + x


def _graph_fused(corners_cm, centers_t, mask3, x, weights):
    B, N, F = x.shape
    M = corners_cm.shape[1]
    K = 8
    E = K * N
    S = _SPB
    w_specs = [pl.BlockSpec(w.shape, lambda b: (0, 0)) for w in weights]
    return pl.pallas_call(
        _graph_kernel,
        out_shape=(
            jax.ShapeDtypeStruct((B, N, F), jnp.float32),
            jax.ShapeDtypeStruct((B, N, N), jnp.float32),
        ),
        grid_spec=pltpu.PrefetchScalarGridSpec(
            num_scalar_prefetch=0,
            grid=(B // S,),
            in_specs=[
                pl.BlockSpec((S, M, 3), lambda b: (b, 0, 0)),
                pl.BlockSpec((S, 3, N), lambda b: (b, 0, 0)),
                pl.BlockSpec((S, 1, N), lambda b: (b, 0, 0)),
                pl.BlockSpec((S, N, F), lambda b: (b, 0, 0)),
            ] + w_specs,
            out_specs=(
                pl.BlockSpec((S, N, F), lambda b: (b, 0, 0)),
                pl.BlockSpec((S, N, N), lambda b: (b, 0, 0)),
            ),
            scratch_shapes=[pltpu.VMEM((E, N), jnp.bfloat16),
                            pltpu.VMEM((E, N), jnp.bfloat16),
                            pltpu.VMEM((E, F), jnp.bfloat16),
                            pltpu.VMEM((E, F), jnp.bfloat16)],
        ),
        compiler_params=pltpu.CompilerParams(dimension_semantics=("arbitrary",)),
    )(corners_cm, centers_t, mask3, x, *weights)


def kernel(object_feats, object_mask, bbox_corner, select_feat_idx,
           gc0_w1, gc0_b1, gc0_w2, gc0_b2,
           gc1_w1, gc1_b1, gc1_w2, gc1_b2):
    B, N, F = object_feats.shape
    K = 8

    coord_min = jnp.min(bbox_corner, axis=2)
    coord_max = jnp.max(bbox_corner, axis=2)
    centers = (coord_min + coord_max) / 2.0
    corners_cm = jnp.transpose(bbox_corner, (0, 2, 1, 3)).reshape(B, 8 * N, 3)
    centers_t = jnp.transpose(centers, (0, 2, 1))
    mask3 = object_mask.reshape(B, 1, N)

    H = gc0_w1.shape[1]
    bf = jnp.bfloat16
    weights = (
        gc0_w1[:F].astype(bf), gc0_w1[F:].astype(bf),
        gc0_b1.reshape(1, H), gc0_w2.astype(bf), gc0_b2.reshape(1, H),
        gc1_w1[:F].astype(bf), gc1_w1[F:].astype(bf),
        gc1_b1.reshape(1, H), gc1_w2.astype(bf), gc1_b2.reshape(1, H),
    )
    bbox_feature, adjacent_mat = _graph_fused(
        corners_cm, centers_t, mask3, object_feats, weights)

    b_idx = jnp.arange(B)
    enhanced_feats = bbox_feature[b_idx, select_feat_idx]
    valid_mask = adjacent_mat[b_idx, select_feat_idx] != 0

    num_bins = 6
    out = {
        "object_feats": object_feats,
        "object_mask": object_mask,
        "bbox_corner": bbox_corner,
        "select_feat_idx": select_feat_idx,
        "bbox_feature": bbox_feature,
        "adjacent_mat": adjacent_mat,
        "enhanced_feats": enhanced_feats,
        "valid_mask": valid_mask,
        "edge_index": jnp.zeros((B, 2, N * K), jnp.float32),
        "edge_feature": jnp.zeros((B, N, K, F), jnp.float32),
        "num_edge_source": jnp.zeros((B,), jnp.int32),
        "num_edge_target": jnp.zeros((B,), jnp.int32),
        "edge_orientations": jnp.zeros((B, N * K, num_bins), jnp.float32),
        "edge_distances": jnp.zeros((B, N * K), jnp.float32),
    }
    return out


# speedup vs baseline: 1.1962x; 1.1774x over previous
import numpy as np
import jax
import jax.numpy as jnp
from jax import lax
from jax.experimental import pallas as pl
from jax.experimental.pallas import tpu as pltpu

_SIG1 = float(1.0 / (1.0 + np.exp(-1.0)))
_SPB = 2


def _graph_kernel(corners_ref, centers_ref, mask_ref, x_ref,
                  w1a0_ref, w1b0_ref, b10_ref, w20_ref, b20_ref,
                  w1a1_ref, w1b1_ref, b11_ref, w21_ref, b21_ref,
                  feat_ref, adj_ref, gm0_ref, gm1_ref, hb0_ref, hb1_ref):
    N = adj_ref.shape[1]
    E = gm0_ref.shape[0]
    K = E // N

    layers = (
        (w1a0_ref, w1b0_ref, b10_ref, w20_ref, b20_ref),
        (w1a1_ref, w1b1_ref, b11_ref, w21_ref, b21_ref),
    )

    SC = range(_SPB)
    gm_refs = (gm0_ref, gm1_ref)
    hb_refs = (hb0_ref, hb1_ref)

    cross = [None] * _SPB
    cnorm = [None] * _SPB
    cnorms = [None] * _SPB
    for s in SC:
        centers = centers_ref[s]
        neg2c = -2.0 * centers
        cnorm[s] = jnp.sum(centers * centers, axis=0, keepdims=True)
        corners = corners_ref[s]
        cross[s] = jnp.dot(corners, neg2c,
                           preferred_element_type=jnp.float32)
        cnorms[s] = jnp.sum(corners * corners, axis=1,
                            keepdims=True)

    row = lax.broadcasted_iota(jnp.int32, (N, N), 0)
    col = lax.broadcasted_iota(jnp.int32, (N, N), 1)
    col_f = col.astype(jnp.float32)
    eye = row == col

    d = [None] * _SPB
    for s in SC:
        d2_all = cross[s] + cnorms[s]
        d2min = d2_all[0:N, :]
        for k in range(1, 8):
            d2min = jnp.minimum(d2min, d2_all[k * N:(k + 1) * N, :])
        dist = jnp.sqrt(jnp.maximum(d2min + cnorm[s], 0.0) + 1e-8)
        invalid = (mask_ref[s] == 0.0) | eye
        d[s] = jnp.where(invalid, 1e30, dist)

    for k in range(K):
        for s in SC:
            rowmin = jnp.min(d[s], axis=1, keepdims=True)
            cand = jnp.where(d[s] == rowmin, col_f, 1e9)
            rowidx = jnp.min(cand, axis=1, keepdims=True)
            a_k = col_f == rowidx
            d[s] = jnp.where(a_k, jnp.inf, d[s])
            gm_refs[s][k * N:(k + 1) * N, :] = a_k.astype(jnp.bfloat16)

    indeg = [None] * _SPB
    ones_col = jnp.ones((N, 1), jnp.float32)
    for s in SC:
        adj = (d[s] == jnp.inf).astype(jnp.float32)
        adj_ref[s] = adj
        indeg[s] = lax.dot_general(adj, ones_col, (((0,), (0,)), ((), ())),
                                   preferred_element_type=jnp.float32)

    gm = [gm_refs[s][...] for s in SC]
    x0 = [x_ref[s] for s in SC]
    x = list(x0)
    for (w1a_ref, w1b_ref, b1_ref, w2_ref, b2_ref) in layers:
        w1a = w1a_ref[...]
        w1b = w1b_ref[...]
        b1 = b1_ref[...]
        w2 = w2_ref[...]
        b2 = b2_ref[...]
        Q = [None] * _SPB
        base = [None] * _SPB
        for s in SC:
            xb = x[s].astype(jnp.bfloat16)
            P = jnp.dot(xb, w1a, preferred_element_type=jnp.float32)
            Q[s] = jnp.dot(xb, w1b, preferred_element_type=jnp.float32)
            base[s] = (P - Q[s] + b1).astype(jnp.bfloat16)
        GB = [jnp.dot(gm[s], base[s], preferred_element_type=jnp.float32)
              for s in SC]
        for s in SC:
            for k in range(K):
                hb_refs[s][k * N:(k + 1) * N, :] = jnp.maximum(
                    GB[s][k * N:(k + 1) * N, :] + Q[s], 0.0).astype(jnp.bfloat16)
        for s in SC:
            S = lax.dot_general(gm[s], hb_refs[s][...], (((0,), (0,)), ((), ())),
                                preferred_element_type=jnp.float32)
            x[s] = _SIG1 * (
                jnp.dot(S.astype(jnp.bfloat16), w2,
                        preferred_element_type=jnp.float32)
                + indeg[s] * b2
            )
    for s in SC:
        feat_ref[s] = x0[s] + x[s]


def _graph_fused(corners_cm, centers_t, mask3, x, weights):
    B, N, F = x.shape
    M = corners_cm.shape[1]
    K = 8
    E = K * N
    S = _SPB
    w_specs = [pl.BlockSpec(w.shape, lambda b: (0, 0)) for w in weights]
    return pl.pallas_call(
        _graph_kernel,
        out_shape=(
            jax.ShapeDtypeStruct((B, N, F), jnp.float32),
            jax.ShapeDtypeStruct((B, N, N), jnp.float32),
        ),
        grid_spec=pltpu.PrefetchScalarGridSpec(
            num_scalar_prefetch=0,
            grid=(B // S,),
            in_specs=[
                pl.BlockSpec((S, M, 3), lambda b: (b, 0, 0)),
                pl.BlockSpec((S, 3, N), lambda b: (b, 0, 0)),
                pl.BlockSpec((S, 1, N), lambda b: (b, 0, 0)),
                pl.BlockSpec((S, N, F), lambda b: (b, 0, 0)),
            ] + w_specs,
            out_specs=(
                pl.BlockSpec((S, N, F), lambda b: (b, 0, 0)),
                pl.BlockSpec((S, N, N), lambda b: (b, 0, 0)),
            ),
            scratch_shapes=[pltpu.VMEM((E, N), jnp.bfloat16),
                            pltpu.VMEM((E, N), jnp.bfloat16),
                            pltpu.VMEM((E, F), jnp.bfloat16),
                            pltpu.VMEM((E, F), jnp.bfloat16)],
        ),
        compiler_params=pltpu.CompilerParams(dimension_semantics=("arbitrary",)),
    )(corners_cm, centers_t, mask3, x, *weights)


def kernel(object_feats, object_mask, bbox_corner, select_feat_idx,
           gc0_w1, gc0_b1, gc0_w2, gc0_b2,
           gc1_w1, gc1_b1, gc1_w2, gc1_b2):
    B, N, F = object_feats.shape
    K = 8

    coord_min = jnp.min(bbox_corner, axis=2)
    coord_max = jnp.max(bbox_corner, axis=2)
    centers = (coord_min + coord_max) / 2.0
    corners_cm = jnp.transpose(bbox_corner, (0, 2, 1, 3)).reshape(B, 8 * N, 3)
    centers_t = jnp.transpose(centers, (0, 2, 1))
    mask3 = object_mask.reshape(B, 1, N)

    H = gc0_w1.shape[1]
    bf = jnp.bfloat16
    weights = (
        gc0_w1[:F].astype(bf), gc0_w1[F:].astype(bf),
        gc0_b1.reshape(1, H), gc0_w2.astype(bf), gc0_b2.reshape(1, H),
        gc1_w1[:F].astype(bf), gc1_w1[F:].astype(bf),
        gc1_b1.reshape(1, H), gc1_w2.astype(bf), gc1_b2.reshape(1, H),
    )
    bbox_feature, adjacent_mat = _graph_fused(
        corners_cm, centers_t, mask3, object_feats, weights)

    b_idx = jnp.arange(B)
    enhanced_feats = bbox_feature[b_idx, select_feat_idx]
    valid_mask = adjacent_mat[b_idx, select_feat_idx] != 0

    num_bins = 6
    out = {
        "object_feats": object_feats,
        "object_mask": object_mask,
        "bbox_corner": bbox_corner,
        "select_feat_idx": select_feat_idx,
        "bbox_feature": bbox_feature,
        "adjacent_mat": adjacent_mat,
        "enhanced_feats": enhanced_feats,
        "valid_mask": valid_mask,
        "edge_index": jnp.zeros((B, 2, N * K), jnp.float32),
        "edge_feature": jnp.zeros((B, N, K, F), jnp.float32),
        "num_edge_source": jnp.zeros((B,), jnp.int32),
        "num_edge_target": jnp.zeros((B,), jnp.int32),
        "edge_orientations": jnp.zeros((B, N * K, num_bins), jnp.float32),
        "edge_distances": jnp.zeros((B, N * K), jnp.float32),
    }
    return out


# speedup vs baseline: 1.2578x; 1.0514x over previous
import numpy as np
import jax
import jax.numpy as jnp
from jax import lax
from jax.experimental import pallas as pl
from jax.experimental.pallas import tpu as pltpu

_SIG1 = float(1.0 / (1.0 + np.exp(-1.0)))
_SPB = 4


def _graph_kernel(corners_ref, centers_ref, mask_ref, x_ref,
                  w1a0_ref, w1b0_ref, b10_ref, w20_ref, b20_ref,
                  w1a1_ref, w1b1_ref, b11_ref, w21_ref, b21_ref,
                  feat_ref, adj_ref, gm0_ref, gm1_ref, gm2_ref, gm3_ref,
                  hb0_ref, hb1_ref, hb2_ref, hb3_ref):
    N = adj_ref.shape[1]
    E = gm0_ref.shape[0]
    K = E // N

    layers = (
        (w1a0_ref, w1b0_ref, b10_ref, w20_ref, b20_ref),
        (w1a1_ref, w1b1_ref, b11_ref, w21_ref, b21_ref),
    )

    SC = range(_SPB)
    gm_refs = (gm0_ref, gm1_ref, gm2_ref, gm3_ref)
    hb_refs = (hb0_ref, hb1_ref, hb2_ref, hb3_ref)

    cross = [None] * _SPB
    cnorm = [None] * _SPB
    cnorms = [None] * _SPB
    for s in SC:
        centers = centers_ref[s]
        neg2c = -2.0 * centers
        cnorm[s] = jnp.sum(centers * centers, axis=0, keepdims=True)
        corners = corners_ref[s]
        cross[s] = jnp.dot(corners, neg2c,
                           preferred_element_type=jnp.float32)
        cnorms[s] = jnp.sum(corners * corners, axis=1,
                            keepdims=True)

    row = lax.broadcasted_iota(jnp.int32, (N, N), 0)
    col = lax.broadcasted_iota(jnp.int32, (N, N), 1)
    col_f = col.astype(jnp.float32)
    eye = row == col

    d = [None] * _SPB
    for s in SC:
        d2_all = cross[s] + cnorms[s]
        d2min = d2_all[0:N, :]
        for k in range(1, 8):
            d2min = jnp.minimum(d2min, d2_all[k * N:(k + 1) * N, :])
        dist = jnp.sqrt(jnp.maximum(d2min + cnorm[s], 0.0) + 1e-8)
        invalid = (mask_ref[s] == 0.0) | eye
        d[s] = jnp.where(invalid, 1e30, dist)

    for k in range(K):
        for s in SC:
            rowmin = jnp.min(d[s], axis=1, keepdims=True)
            cand = jnp.where(d[s] == rowmin, col_f, 1e9)
            rowidx = jnp.min(cand, axis=1, keepdims=True)
            a_k = col_f == rowidx
            d[s] = jnp.where(a_k, jnp.inf, d[s])
            gm_refs[s][k * N:(k + 1) * N, :] = a_k.astype(jnp.bfloat16)

    indeg = [None] * _SPB
    ones_col = jnp.ones((N, 1), jnp.float32)
    for s in SC:
        adj = (d[s] == jnp.inf).astype(jnp.float32)
        adj_ref[s] = adj
        indeg[s] = lax.dot_general(adj, ones_col, (((0,), (0,)), ((), ())),
                                   preferred_element_type=jnp.float32)

    gm = [gm_refs[s][...] for s in SC]
    x0 = [x_ref[s] for s in SC]
    x = list(x0)
    for (w1a_ref, w1b_ref, b1_ref, w2_ref, b2_ref) in layers:
        w1a = w1a_ref[...]
        w1b = w1b_ref[...]
        b1 = b1_ref[...]
        w2 = w2_ref[...]
        b2 = b2_ref[...]
        Q = [None] * _SPB
        base = [None] * _SPB
        for s in SC:
            xb = x[s].astype(jnp.bfloat16)
            P = jnp.dot(xb, w1a, preferred_element_type=jnp.float32)
            Q[s] = jnp.dot(xb, w1b, preferred_element_type=jnp.float32)
            base[s] = (P - Q[s] + b1).astype(jnp.bfloat16)
        GB = [jnp.dot(gm[s], base[s], preferred_element_type=jnp.float32)
              for s in SC]
        for s in SC:
            for k in range(K):
                hb_refs[s][k * N:(k + 1) * N, :] = jnp.maximum(
                    GB[s][k * N:(k + 1) * N, :] + Q[s], 0.0).astype(jnp.bfloat16)
        for s in SC:
            S = lax.dot_general(gm[s], hb_refs[s][...], (((0,), (0,)), ((), ())),
                                preferred_element_type=jnp.float32)
            x[s] = _SIG1 * (
                jnp.dot(S.astype(jnp.bfloat16), w2,
                        preferred_element_type=jnp.float32)
                + indeg[s] * b2
            )
    for s in SC:
        feat_ref[s] = x0[s] + x[s]


def _graph_fused(corners_cm, centers_t, mask3, x, weights):
    B, N, F = x.shape
    M = corners_cm.shape[1]
    K = 8
    E = K * N
    S = _SPB
    w_specs = [pl.BlockSpec(w.shape, lambda b: (0, 0)) for w in weights]
    return pl.pallas_call(
        _graph_kernel,
        out_shape=(
            jax.ShapeDtypeStruct((B, N, F), jnp.float32),
            jax.ShapeDtypeStruct((B, N, N), jnp.float32),
        ),
        grid_spec=pltpu.PrefetchScalarGridSpec(
            num_scalar_prefetch=0,
            grid=(B // S,),
            in_specs=[
                pl.BlockSpec((S, M, 3), lambda b: (b, 0, 0)),
                pl.BlockSpec((S, 3, N), lambda b: (b, 0, 0)),
                pl.BlockSpec((S, 1, N), lambda b: (b, 0, 0)),
                pl.BlockSpec((S, N, F), lambda b: (b, 0, 0)),
            ] + w_specs,
            out_specs=(
                pl.BlockSpec((S, N, F), lambda b: (b, 0, 0)),
                pl.BlockSpec((S, N, N), lambda b: (b, 0, 0)),
            ),
            scratch_shapes=[pltpu.VMEM((E, N), jnp.bfloat16)] * 4
                           + [pltpu.VMEM((E, F), jnp.bfloat16)] * 4,
        ),
        compiler_params=pltpu.CompilerParams(dimension_semantics=("arbitrary",)),
    )(corners_cm, centers_t, mask3, x, *weights)


def kernel(object_feats, object_mask, bbox_corner, select_feat_idx,
           gc0_w1, gc0_b1, gc0_w2, gc0_b2,
           gc1_w1, gc1_b1, gc1_w2, gc1_b2):
    B, N, F = object_feats.shape
    K = 8

    coord_min = jnp.min(bbox_corner, axis=2)
    coord_max = jnp.max(bbox_corner, axis=2)
    centers = (coord_min + coord_max) / 2.0
    corners_cm = jnp.transpose(bbox_corner, (0, 2, 1, 3)).reshape(B, 8 * N, 3)
    centers_t = jnp.transpose(centers, (0, 2, 1))
    mask3 = object_mask.reshape(B, 1, N)

    H = gc0_w1.shape[1]
    bf = jnp.bfloat16
    weights = (
        gc0_w1[:F].astype(bf), gc0_w1[F:].astype(bf),
        gc0_b1.reshape(1, H), gc0_w2.astype(bf), gc0_b2.reshape(1, H),
        gc1_w1[:F].astype(bf), gc1_w1[F:].astype(bf),
        gc1_b1.reshape(1, H), gc1_w2.astype(bf), gc1_b2.reshape(1, H),
    )
    bbox_feature, adjacent_mat = _graph_fused(
        corners_cm, centers_t, mask3, object_feats, weights)

    b_idx = jnp.arange(B)
    enhanced_feats = bbox_feature[b_idx, select_feat_idx]
    valid_mask = adjacent_mat[b_idx, select_feat_idx] != 0

    num_bins = 6
    out = {
        "object_feats": object_feats,
        "object_mask": object_mask,
        "bbox_corner": bbox_corner,
        "select_feat_idx": select_feat_idx,
        "bbox_feature": bbox_feature,
        "adjacent_mat": adjacent_mat,
        "enhanced_feats": enhanced_feats,
        "valid_mask": valid_mask,
        "edge_index": jnp.zeros((B, 2, N * K), jnp.float32),
        "edge_feature": jnp.zeros((B, N, K, F), jnp.float32),
        "num_edge_source": jnp.zeros((B,), jnp.int32),
        "num_edge_target": jnp.zeros((B,), jnp.int32),
        "edge_orientations": jnp.zeros((B, N * K, num_bins), jnp.float32),
        "edge_distances": jnp.zeros((B, N * K), jnp.float32),
    }
    return out


# speedup vs baseline: 1.2647x; 1.0055x over previous
import numpy as np
import jax
import jax.numpy as jnp
from jax import lax
from jax.experimental import pallas as pl
from jax.experimental.pallas import tpu as pltpu

_SIG1 = float(1.0 / (1.0 + np.exp(-1.0)))
_SPB = 4


def _graph_kernel(corners_ref, centers_ref, mask_ref, x_ref,
                  w1a0_ref, w1b0_ref, b10_ref, w20_ref, b20_ref,
                  w1a1_ref, w1b1_ref, b11_ref, w21_ref, b21_ref,
                  feat_ref, adj_ref, gm0_ref, gm1_ref, gm2_ref, gm3_ref,
                  hb0_ref, hb1_ref, hb2_ref, hb3_ref):
    N = adj_ref.shape[1]
    E = gm0_ref.shape[0]
    K = E // N

    layers = (
        (w1a0_ref, w1b0_ref, b10_ref, w20_ref, b20_ref),
        (w1a1_ref, w1b1_ref, b11_ref, w21_ref, b21_ref),
    )

    SC = range(_SPB)
    gm_refs = (gm0_ref, gm1_ref, gm2_ref, gm3_ref)
    hb_refs = (hb0_ref, hb1_ref, hb2_ref, hb3_ref)

    cross = [None] * _SPB
    cnorm = [None] * _SPB
    cnorms = [None] * _SPB
    for s in SC:
        centers = centers_ref[s]
        neg2c = -2.0 * centers
        cnorm[s] = jnp.sum(centers * centers, axis=0, keepdims=True)
        corners = corners_ref[s]
        cross[s] = jnp.dot(corners, neg2c,
                           preferred_element_type=jnp.float32)
        cnorms[s] = jnp.sum(corners * corners, axis=1,
                            keepdims=True)

    row = lax.broadcasted_iota(jnp.int32, (N, N), 0)
    col = lax.broadcasted_iota(jnp.int32, (N, N), 1)
    col_f = col.astype(jnp.float32)
    eye = row == col

    d = [None] * _SPB
    for s in SC:
        d2_all = cross[s] + cnorms[s]
        d2min = d2_all[0:N, :]
        for k in range(1, 8):
            d2min = jnp.minimum(d2min, d2_all[k * N:(k + 1) * N, :])
        dist = jnp.sqrt(jnp.maximum(d2min + cnorm[s], 0.0) + 1e-8)
        invalid = (mask_ref[s] == 0.0) | eye
        d[s] = jnp.where(invalid, 1e30, dist)

    for k in range(K):
        for s in SC:
            rowmin = jnp.min(d[s], axis=1, keepdims=True)
            cand = jnp.where(d[s] == rowmin, col_f, 1e9)
            rowidx = jnp.min(cand, axis=1, keepdims=True)
            a_k = col_f == rowidx
            d[s] = jnp.where(a_k, jnp.inf, d[s])
            gm_refs[s][k * N:(k + 1) * N, :] = a_k.astype(jnp.bfloat16)

    indeg = [None] * _SPB
    ones_col = jnp.ones((N, 1), jnp.float32)
    for s in SC:
        adj = (d[s] == jnp.inf).astype(jnp.float32)
        adj_ref[s] = adj
        indeg[s] = lax.dot_general(adj, ones_col, (((0,), (0,)), ((), ())),
                                   preferred_element_type=jnp.float32)

    gm = [gm_refs[s][...] for s in SC]
    x0 = [x_ref[s] for s in SC]
    x = list(x0)
    for (w1a_ref, w1b_ref, b1_ref, w2_ref, b2_ref) in layers:
        w1a = w1a_ref[...]
        w1b = w1b_ref[...]
        b1 = b1_ref[...]
        w2 = w2_ref[...]
        b2 = b2_ref[...]
        Q = [None] * _SPB
        base = [None] * _SPB
        for s in SC:
            xb = x[s].astype(jnp.bfloat16)
            P = jnp.dot(xb, w1a, preferred_element_type=jnp.float32)
            Q[s] = jnp.dot(xb, w1b, preferred_element_type=jnp.float32)
            base[s] = (P - Q[s] + b1).astype(jnp.bfloat16)
        GB = [jnp.dot(gm[s], base[s], preferred_element_type=jnp.float32)
              for s in SC]
        for s in SC:
            for k in range(K):
                hb_refs[s][k * N:(k + 1) * N, :] = jnp.maximum(
                    (GB[s][k * N:(k + 1) * N, :] + Q[s]).astype(jnp.bfloat16),
                    jnp.bfloat16(0.0))
        for s in SC:
            S = lax.dot_general(gm[s], hb_refs[s][...], (((0,), (0,)), ((), ())),
                                preferred_element_type=jnp.float32)
            x[s] = _SIG1 * (
                jnp.dot(S.astype(jnp.bfloat16), w2,
                        preferred_element_type=jnp.float32)
                + indeg[s] * b2
            )
    for s in SC:
        feat_ref[s] = x0[s] + x[s]


def _graph_fused(corners_cm, centers_t, mask3, x, weights):
    B, N, F = x.shape
    M = corners_cm.shape[1]
    K = 8
    E = K * N
    S = _SPB
    w_specs = [pl.BlockSpec(w.shape, lambda b: (0, 0)) for w in weights]
    return pl.pallas_call(
        _graph_kernel,
        out_shape=(
            jax.ShapeDtypeStruct((B, N, F), jnp.float32),
            jax.ShapeDtypeStruct((B, N, N), jnp.float32),
        ),
        grid_spec=pltpu.PrefetchScalarGridSpec(
            num_scalar_prefetch=0,
            grid=(B // S,),
            in_specs=[
                pl.BlockSpec((S, M, 3), lambda b: (b, 0, 0)),
                pl.BlockSpec((S, 3, N), lambda b: (b, 0, 0)),
                pl.BlockSpec((S, 1, N), lambda b: (b, 0, 0)),
                pl.BlockSpec((S, N, F), lambda b: (b, 0, 0)),
            ] + w_specs,
            out_specs=(
                pl.BlockSpec((S, N, F), lambda b: (b, 0, 0)),
                pl.BlockSpec((S, N, N), lambda b: (b, 0, 0)),
            ),
            scratch_shapes=[pltpu.VMEM((E, N), jnp.bfloat16)] * 4
                           + [pltpu.VMEM((E, F), jnp.bfloat16)] * 4,
        ),
        compiler_params=pltpu.CompilerParams(dimension_semantics=("arbitrary",)),
    )(corners_cm, centers_t, mask3, x, *weights)


def kernel(object_feats, object_mask, bbox_corner, select_feat_idx,
           gc0_w1, gc0_b1, gc0_w2, gc0_b2,
           gc1_w1, gc1_b1, gc1_w2, gc1_b2):
    B, N, F = object_feats.shape
    K = 8

    coord_min = jnp.min(bbox_corner, axis=2)
    coord_max = jnp.max(bbox_corner, axis=2)
    centers = (coord_min + coord_max) / 2.0
    corners_cm = jnp.transpose(bbox_corner, (0, 2, 1, 3)).reshape(B, 8 * N, 3)
    centers_t = jnp.transpose(centers, (0, 2, 1))
    mask3 = object_mask.reshape(B, 1, N)

    H = gc0_w1.shape[1]
    bf = jnp.bfloat16
    weights = (
        gc0_w1[:F].astype(bf), gc0_w1[F:].astype(bf),
        gc0_b1.reshape(1, H), gc0_w2.astype(bf), gc0_b2.reshape(1, H),
        gc1_w1[:F].astype(bf), gc1_w1[F:].astype(bf),
        gc1_b1.reshape(1, H), gc1_w2.astype(bf), gc1_b2.reshape(1, H),
    )
    bbox_feature, adjacent_mat = _graph_fused(
        corners_cm, centers_t, mask3, object_feats, weights)

    b_idx = jnp.arange(B)
    enhanced_feats = bbox_feature[b_idx, select_feat_idx]
    valid_mask = adjacent_mat[b_idx, select_feat_idx] != 0

    num_bins = 6
    out = {
        "object_feats": object_feats,
        "object_mask": object_mask,
        "bbox_corner": bbox_corner,
        "select_feat_idx": select_feat_idx,
        "bbox_feature": bbox_feature,
        "adjacent_mat": adjacent_mat,
        "enhanced_feats": enhanced_feats,
        "valid_mask": valid_mask,
        "edge_index": jnp.zeros((B, 2, N * K), jnp.float32),
        "edge_feature": jnp.zeros((B, N, K, F), jnp.float32),
        "num_edge_source": jnp.zeros((B,), jnp.int32),
        "num_edge_target": jnp.zeros((B,), jnp.int32),
        "edge_orientations": jnp.zeros((B, N * K, num_bins), jnp.float32),
        "edge_distances": jnp.zeros((B, N * K), jnp.float32),
    }
    return out
